# baseline (speedup 1.0000x reference)
import functools

import jax
import jax.numpy as jnp
from jax import lax
from jax.experimental import pallas as pl
from jax.experimental.pallas import tpu as pltpu
from jax.experimental.pallas import tpu_sc as plsc

TM = 512
NW = 32
CH = 32


def _cumsum0_excl(v):
    n = v.shape[0]
    total = v
    k = 1
    while k < n:
        shifted = jnp.concatenate(
            [jnp.zeros((k, v.shape[1]), v.dtype), total[:-k]], axis=0)
        total = total + shifted
        k *= 2
    return total - v


def _gating_body(x_ref, wg_ref, bg_ref,
                 pa_ref, pb_ref, wa_ref, wb_ref, te_ref, tv_ref, xb_ref,
                 ll_ref, il_ref):
    B = x_ref.shape[0]
    E = wg_ref.shape[0]
    NT = te_ref.shape[0]
    s = lax.dot_general(x_ref[...], wg_ref[...], (((1,), (1,)), ((), ())),
                        preferred_element_type=jnp.float32) + bg_ref[...]
    ids = lax.broadcasted_iota(jnp.int32, s.shape, 1)
    m1 = jnp.max(s, axis=1, keepdims=True)
    a1 = jnp.min(jnp.where(s == m1, ids, E), axis=1, keepdims=True)
    s2 = jnp.where(ids == a1, -jnp.inf, s)
    m2 = jnp.max(s2, axis=1, keepdims=True)
    a2 = jnp.min(jnp.where(s2 == m2, ids, E), axis=1, keepdims=True)
    t = jnp.exp(m2 - m1)
    lanes = jnp.ones((1, 16), jnp.float32)
    wa_ref[...] = (1.0 / (1.0 + t)) * lanes
    wb_ref[...] = (t / (1.0 + t)) * lanes
    oh1 = (ids == a1).astype(jnp.float32)
    oh2 = (ids == a2).astype(jnp.float32)
    c12 = _cumsum0_excl(jnp.concatenate([oh1, oh2], axis=1))
    c1, c2 = c12[:, :E], c12[:, E:]
    cnt1 = jnp.sum(oh1, axis=0, keepdims=True)
    cnt2 = jnp.sum(oh2, axis=0, keepdims=True)
    counts = cnt1 + cnt2
    padded = jnp.ceil(counts / TM) * TM
    offpad = jnp.zeros_like(padded)
    acc = padded
    k = 1
    while k < E:
        offpad = offpad + jnp.concatenate(
            [jnp.zeros((1, k), jnp.float32), acc[:, :-k]], axis=1)
        acc = jnp.concatenate(
            [jnp.zeros((1, k), jnp.float32), acc[:, :-k]], axis=1) + acc
        k *= 2
    pa = jnp.sum(oh1 * (offpad + c1), axis=1, keepdims=True)
    pb = jnp.sum(oh2 * (offpad + cnt1 + c2), axis=1, keepdims=True)
    pa_ref[...] = pa.astype(jnp.int32)[:, 0]
    pb_ref[...] = pb.astype(jnp.int32)[:, 0]
    cumtiles = (offpad + padded) / TM
    mrow = lax.broadcasted_iota(jnp.int32, (NT, E), 0).astype(jnp.float32)
    te = jnp.sum((mrow >= cumtiles).astype(jnp.float32), axis=1, keepdims=True)
    total_tiles = jnp.sum(padded, axis=1, keepdims=True) / TM
    eids = lax.broadcasted_iota(jnp.int32, (1, E), 1).astype(jnp.float32)
    laste = jnp.max(jnp.where(counts > 0, eids, 0.0), axis=1, keepdims=True)
    mcol = lax.broadcasted_iota(jnp.int32, (NT, 1), 0).astype(jnp.float32)
    tvalid = (mcol < total_tiles).astype(jnp.float32)
    te = jnp.where(tvalid > 0, jnp.minimum(te, E - 1), laste)
    te_ref[...] = te.astype(jnp.int32)[:, 0]
    tv_ref[...] = tvalid.astype(jnp.int32)[:, 0]
    xb_ref[...] = jnp.minimum(mcol, total_tiles - 1).astype(jnp.int32)[:, 0]
    lmean = jnp.sum(counts, axis=1, keepdims=True) / E
    ldev = counts - lmean
    lvar = jnp.sum(ldev * ldev, axis=1, keepdims=True) / (E - 1)
    ll_ref[...] = lvar / (E * (B / E))
    p = jnp.exp(s - m1)
    p = p / jnp.sum(p, axis=1, keepdims=True)
    imp = jnp.sum(p, axis=0, keepdims=True)
    imean = jnp.sum(imp, axis=1, keepdims=True) / E
    idev = imp - imean
    ivar = jnp.sum(idev * idev, axis=1, keepdims=True) / (E - 1)
    il_ref[...] = ivar / (imean + 1e-8)


def _gating(x, Wg, bg, NT):
    B, D = x.shape
    E = Wg.shape[0]
    outs = [
        jax.ShapeDtypeStruct((B,), jnp.int32),
        jax.ShapeDtypeStruct((B,), jnp.int32),
        jax.ShapeDtypeStruct((B, 16), jnp.float32),
        jax.ShapeDtypeStruct((B, 16), jnp.float32),
        jax.ShapeDtypeStruct((NT,), jnp.int32),
        jax.ShapeDtypeStruct((NT,), jnp.int32),
        jax.ShapeDtypeStruct((NT,), jnp.int32),
        jax.ShapeDtypeStruct((1, 1), jnp.float32),
        jax.ShapeDtypeStruct((1, 1), jnp.float32),
    ]
    return pl.pallas_call(_gating_body, out_shape=outs)(x, Wg, bg.reshape(1, E))


def _gmm_body(te_ref, tv_ref, xb_ref, xs_ref, w1_ref, b1_ref, w2_ref, b2_ref,
              out_ref):
    i = pl.program_id(0)

    @pl.when(tv_ref[i] == 1)
    def _():
        h = lax.dot_general(xs_ref[...], w1_ref[0], (((1,), (1,)), ((), ())),
                            preferred_element_type=jnp.float32)
        h = jnp.maximum(h + b1_ref[0], 0.0)
        o = lax.dot_general(h, w2_ref[0], (((1,), (1,)), ((), ())),
                            preferred_element_type=jnp.float32)
        out_ref[...] = o + b2_ref[0]


def _gmm(tile_expert, tile_valid, xs_blk, xs, W1, b1, W2, b2):
    NPAD, D = xs.shape
    NT = NPAD // TM
    grid_spec = pltpu.PrefetchScalarGridSpec(
        num_scalar_prefetch=3,
        grid=(NT,),
        in_specs=[
            pl.BlockSpec((TM, D), lambda i, te, tv, xb: (xb[i], 0)),
            pl.BlockSpec((1, D, D), lambda i, te, tv, xb: (te[i], 0, 0)),
            pl.BlockSpec((1, 1, D), lambda i, te, tv, xb: (te[i], 0, 0)),
            pl.BlockSpec((1, D, D), lambda i, te, tv, xb: (te[i], 0, 0)),
            pl.BlockSpec((1, 1, D), lambda i, te, tv, xb: (te[i], 0, 0)),
        ],
        out_specs=pl.BlockSpec((TM, D), lambda i, te, tv, xb: (xb[i], 0)),
    )
    return pl.pallas_call(
        _gmm_body,
        grid_spec=grid_spec,
        out_shape=jax.ShapeDtypeStruct((NPAD, D), jnp.float32),
    )(tile_expert, tile_valid, xs_blk, xs, W1, b1.reshape(b1.shape[0], 1, D),
      W2, b2.reshape(b2.shape[0], 1, D))


def _sc_dispatch(x, pa, pb, NPAD):
    B, D = x.shape
    TPW = B // NW
    n_ch = TPW // CH
    mesh = plsc.VectorSubcoreMesh(core_axis_name="c", subcore_axis_name="s")

    @functools.partial(
        pl.kernel, mesh=mesh,
        out_type=jax.ShapeDtypeStruct((NPAD, D), jnp.float32),
        scratch_types=[
            pltpu.VMEM((2, 2, CH), jnp.int32),
            pltpu.VMEM((2, CH, D), jnp.float32),
            pltpu.SemaphoreType.DMA,
            pltpu.SemaphoreType.DMA,
            pltpu.SemaphoreType.DMA,
        ],
    )
    def k(x_hbm, pa_hbm, pb_hbm, out_hbm, idx_v, xv, semx, sem1, sem2):
        info = plsc.get_sparse_core_info()
        wid = lax.axis_index("s") * info.num_cores + lax.axis_index("c")
        base = wid * TPW

        def load(c, slot):
            return pltpu.async_copy(
                x_hbm.at[pl.ds(base + c * CH, CH)], xv.at[slot], semx)

        cpx = load(0, 0)
        scat = None
        for c in range(n_ch):
            slot = c % 2
            cpx.wait()
            pltpu.sync_copy(pa_hbm.at[pl.ds(base + c * CH, CH)],
                            idx_v.at[slot, 0])
            pltpu.sync_copy(pb_hbm.at[pl.ds(base + c * CH, CH)],
                            idx_v.at[slot, 1])
            if scat is not None:
                scat[0].wait()
                scat[1].wait()
            scat = (
                pltpu.async_copy(xv.at[slot],
                                 out_hbm.at[idx_v.at[slot, 0]], sem1),
                pltpu.async_copy(xv.at[slot],
                                 out_hbm.at[idx_v.at[slot, 1]], sem2),
            )
            if c + 1 < n_ch:
                cpx = load(c + 1, (c + 1) % 2)
        scat[0].wait()
        scat[1].wait()

    return k(x, pa, pb)


def _sc_combine(o, slotA, slotB, wA, wB):
    NPAD, D = o.shape
    B = slotA.shape[0]
    TPW = B // NW
    CC = 16
    n_ch = TPW // CC
    U = 4
    mesh = plsc.VectorSubcoreMesh(core_axis_name="c", subcore_axis_name="s")

    @functools.partial(
        pl.kernel, mesh=mesh,
        out_type=jax.ShapeDtypeStruct((B, D), jnp.float32),
        scratch_types=[
            pltpu.VMEM((TPW,), jnp.int32),
            pltpu.VMEM((TPW,), jnp.int32),
            pltpu.VMEM((TPW, 16), jnp.float32),
            pltpu.VMEM((TPW, 16), jnp.float32),
            pltpu.VMEM((2, CC, D), jnp.float32),
            pltpu.VMEM((2, CC, D), jnp.float32),
            pltpu.SemaphoreType.DMA,
            pltpu.SemaphoreType.DMA,
        ],
    )
    def k(o_hbm, ia_hbm, ib_hbm, wa_hbm, wb_hbm, out_hbm,
          ia_v, ib_v, wa_v, wb_v, bufa, bufb, sema, semb):
        info = plsc.get_sparse_core_info()
        wid = lax.axis_index("s") * info.num_cores + lax.axis_index("c")
        base = wid * TPW
        pltpu.sync_copy(ia_hbm.at[pl.ds(base, TPW)], ia_v)
        pltpu.sync_copy(ib_hbm.at[pl.ds(base, TPW)], ib_v)
        pltpu.sync_copy(wa_hbm.at[pl.ds(base, TPW)], wa_v)
        pltpu.sync_copy(wb_hbm.at[pl.ds(base, TPW)], wb_v)

        def fire(c, slot):
            return (
                pltpu.async_copy(o_hbm.at[ia_v.at[pl.ds(c * CC, CC)]],
                                 bufa.at[slot], sema),
                pltpu.async_copy(o_hbm.at[ib_v.at[pl.ds(c * CC, CC)]],
                                 bufb.at[slot], semb),
            )

        cps = fire(0, 0)
        for c in range(n_ch):
            slot = c % 2
            cps[0].wait()
            cps[1].wait()
            if c + 1 < n_ch:
                cps = fire(c + 1, (c + 1) % 2)

            def add_row(i, carry2):
                wa = wa_v[c * CC + i, pl.ds(0, 16)]
                wb = wb_v[c * CC + i, pl.ds(0, 16)]
                for j in range(D // 16):
                    sl = pl.ds(16 * j, 16)
                    bufa[slot, i, sl] = (wa * bufa[slot, i, sl]
                                         + wb * bufb[slot, i, sl])
                return carry2

            lax.fori_loop(0, CC, add_row, 0)
            pltpu.sync_copy(bufa.at[slot], out_hbm.at[pl.ds(base + c * CC, CC)])

        return None

    return k(o, slotA, slotB, wA, wB)


def kernel(x, Wg, bg, W1, b1, W2, b2):
    B, D = x.shape
    E = Wg.shape[0]
    BK = 2 * B
    NT = (BK + E * (TM - 1) + TM - 1) // TM
    NPAD = NT * TM

    pa, pb, wa, wb, te, tv, xb, ll, il = _gating(x, Wg, bg, NT)

    xs = _sc_dispatch(x, pa, pb, NPAD)
    o = _gmm(te, tv, xb, xs, W1, b1, W2, b2)
    out = _sc_combine(o, pa, pb, wa, wb)

    return out, ll[0, 0], il[0, 0]


# speedup vs baseline: 1.0791x; 1.0791x over previous
import functools

import jax
import jax.numpy as jnp
from jax import lax
from jax.experimental import pallas as pl
from jax.experimental.pallas import tpu as pltpu
from jax.experimental.pallas import tpu_sc as plsc

TM = 512
NW = 32
CH = 32


def _cumsum0_excl(v):
    n = v.shape[0]
    total = v
    k = 1
    while k < n:
        shifted = jnp.concatenate(
            [jnp.zeros((k, v.shape[1]), v.dtype), total[:-k]], axis=0)
        total = total + shifted
        k *= 2
    return total - v


def _gating_body(x_ref, wg_ref, bg_ref,
                 pa_ref, pb_ref, wa_ref, wb_ref, te_ref, tv_ref, xb_ref,
                 ll_ref, il_ref):
    B = x_ref.shape[0]
    E = wg_ref.shape[0]
    NT = te_ref.shape[0]
    s = lax.dot_general(x_ref[...], wg_ref[...], (((1,), (1,)), ((), ())),
                        preferred_element_type=jnp.float32) + bg_ref[...]
    ids = lax.broadcasted_iota(jnp.int32, s.shape, 1)
    m1 = jnp.max(s, axis=1, keepdims=True)
    a1 = jnp.min(jnp.where(s == m1, ids, E), axis=1, keepdims=True)
    s2 = jnp.where(ids == a1, -jnp.inf, s)
    m2 = jnp.max(s2, axis=1, keepdims=True)
    a2 = jnp.min(jnp.where(s2 == m2, ids, E), axis=1, keepdims=True)
    t = jnp.exp(m2 - m1)
    lanes = jnp.ones((1, 16), jnp.float32)
    wa_ref[...] = (1.0 / (1.0 + t)) * lanes
    wb_ref[...] = (t / (1.0 + t)) * lanes
    oh1 = (ids == a1).astype(jnp.float32)
    oh2 = (ids == a2).astype(jnp.float32)
    c12 = _cumsum0_excl(jnp.concatenate([oh1, oh2], axis=1))
    c1, c2 = c12[:, :E], c12[:, E:]
    cnt1 = jnp.sum(oh1, axis=0, keepdims=True)
    cnt2 = jnp.sum(oh2, axis=0, keepdims=True)
    counts = cnt1 + cnt2
    padded = jnp.ceil(counts / TM) * TM
    offpad = jnp.zeros_like(padded)
    acc = padded
    k = 1
    while k < E:
        offpad = offpad + jnp.concatenate(
            [jnp.zeros((1, k), jnp.float32), acc[:, :-k]], axis=1)
        acc = jnp.concatenate(
            [jnp.zeros((1, k), jnp.float32), acc[:, :-k]], axis=1) + acc
        k *= 2
    pa = jnp.sum(oh1 * (offpad + c1), axis=1, keepdims=True)
    pb = jnp.sum(oh2 * (offpad + cnt1 + c2), axis=1, keepdims=True)
    pa_ref[...] = pa.astype(jnp.int32)[:, 0]
    pb_ref[...] = pb.astype(jnp.int32)[:, 0]
    cumtiles = (offpad + padded) / TM
    mrow = lax.broadcasted_iota(jnp.int32, (NT, E), 0).astype(jnp.float32)
    te = jnp.sum((mrow >= cumtiles).astype(jnp.float32), axis=1, keepdims=True)
    total_tiles = jnp.sum(padded, axis=1, keepdims=True) / TM
    eids = lax.broadcasted_iota(jnp.int32, (1, E), 1).astype(jnp.float32)
    laste = jnp.max(jnp.where(counts > 0, eids, 0.0), axis=1, keepdims=True)
    mcol = lax.broadcasted_iota(jnp.int32, (NT, 1), 0).astype(jnp.float32)
    tvalid = (mcol < total_tiles).astype(jnp.float32)
    te = jnp.where(tvalid > 0, jnp.minimum(te, E - 1), laste)
    te_ref[...] = te.astype(jnp.int32)[:, 0]
    tv_ref[...] = tvalid.astype(jnp.int32)[:, 0]
    xb_ref[...] = jnp.minimum(mcol, total_tiles - 1).astype(jnp.int32)[:, 0]
    lmean = jnp.sum(counts, axis=1, keepdims=True) / E
    ldev = counts - lmean
    lvar = jnp.sum(ldev * ldev, axis=1, keepdims=True) / (E - 1)
    ll_ref[...] = lvar / (E * (B / E))
    p = jnp.exp(s - m1)
    p = p / jnp.sum(p, axis=1, keepdims=True)
    imp = jnp.sum(p, axis=0, keepdims=True)
    imean = jnp.sum(imp, axis=1, keepdims=True) / E
    idev = imp - imean
    ivar = jnp.sum(idev * idev, axis=1, keepdims=True) / (E - 1)
    il_ref[...] = ivar / (imean + 1e-8)


def _gating(x, Wg, bg, NT):
    B, D = x.shape
    E = Wg.shape[0]
    outs = [
        jax.ShapeDtypeStruct((B,), jnp.int32),
        jax.ShapeDtypeStruct((B,), jnp.int32),
        jax.ShapeDtypeStruct((B, 16), jnp.float32),
        jax.ShapeDtypeStruct((B, 16), jnp.float32),
        jax.ShapeDtypeStruct((NT,), jnp.int32),
        jax.ShapeDtypeStruct((NT,), jnp.int32),
        jax.ShapeDtypeStruct((NT,), jnp.int32),
        jax.ShapeDtypeStruct((1, 1), jnp.float32),
        jax.ShapeDtypeStruct((1, 1), jnp.float32),
    ]
    return pl.pallas_call(_gating_body, out_shape=outs)(x, Wg, bg.reshape(1, E))


def _gmm_body(te_ref, tv_ref, xb_ref, xs_ref, w1_ref, b1_ref, w2_ref, b2_ref,
              out_ref):
    i = pl.program_id(0)

    @pl.when(tv_ref[i] == 1)
    def _():
        h = lax.dot_general(xs_ref[...], w1_ref[0], (((1,), (1,)), ((), ())),
                            preferred_element_type=jnp.float32)
        h = jnp.maximum(h + b1_ref[0], 0.0)
        o = lax.dot_general(h, w2_ref[0], (((1,), (1,)), ((), ())),
                            preferred_element_type=jnp.float32)
        out_ref[...] = o + b2_ref[0]


def _gmm(tile_expert, tile_valid, xs_blk, xs, W1, b1, W2, b2):
    NPAD, D = xs.shape
    NT = NPAD // TM
    grid_spec = pltpu.PrefetchScalarGridSpec(
        num_scalar_prefetch=3,
        grid=(NT,),
        in_specs=[
            pl.BlockSpec((TM, D), lambda i, te, tv, xb: (xb[i], 0)),
            pl.BlockSpec((1, D, D), lambda i, te, tv, xb: (te[i], 0, 0)),
            pl.BlockSpec((1, 1, D), lambda i, te, tv, xb: (te[i], 0, 0)),
            pl.BlockSpec((1, D, D), lambda i, te, tv, xb: (te[i], 0, 0)),
            pl.BlockSpec((1, 1, D), lambda i, te, tv, xb: (te[i], 0, 0)),
        ],
        out_specs=pl.BlockSpec((TM, D), lambda i, te, tv, xb: (xb[i], 0)),
    )
    return pl.pallas_call(
        _gmm_body,
        grid_spec=grid_spec,
        out_shape=jax.ShapeDtypeStruct((NPAD, D), jnp.float32),
    )(tile_expert, tile_valid, xs_blk, xs, W1, b1.reshape(b1.shape[0], 1, D),
      W2, b2.reshape(b2.shape[0], 1, D))


def _sc_dispatch(x, pa, pb, NPAD):
    B, D = x.shape
    TPW = B // NW
    n_ch = TPW // CH
    mesh = plsc.VectorSubcoreMesh(core_axis_name="c", subcore_axis_name="s")

    @functools.partial(
        pl.kernel, mesh=mesh,
        out_type=jax.ShapeDtypeStruct((NPAD, D), jnp.float32),
        scratch_types=[
            pltpu.VMEM((2, 2, CH), jnp.int32),
            pltpu.VMEM((2, CH, D), jnp.float32),
            pltpu.SemaphoreType.DMA,
            pltpu.SemaphoreType.DMA,
            pltpu.SemaphoreType.DMA,
        ],
    )
    def k(x_hbm, pa_hbm, pb_hbm, out_hbm, idx_v, xv, semx, sem1, sem2):
        info = plsc.get_sparse_core_info()
        wid = lax.axis_index("s") * info.num_cores + lax.axis_index("c")
        base = wid * TPW

        def load(c, slot):
            return pltpu.async_copy(
                x_hbm.at[pl.ds(base + c * CH, CH)], xv.at[slot], semx)

        cpx = load(0, 0)
        scat = None
        for c in range(n_ch):
            slot = c % 2
            cpx.wait()
            pltpu.sync_copy(pa_hbm.at[pl.ds(base + c * CH, CH)],
                            idx_v.at[slot, 0])
            pltpu.sync_copy(pb_hbm.at[pl.ds(base + c * CH, CH)],
                            idx_v.at[slot, 1])
            if scat is not None:
                scat[0].wait()
                scat[1].wait()
            scat = (
                pltpu.async_copy(xv.at[slot],
                                 out_hbm.at[idx_v.at[slot, 0]], sem1),
                pltpu.async_copy(xv.at[slot],
                                 out_hbm.at[idx_v.at[slot, 1]], sem2),
            )
            if c + 1 < n_ch:
                cpx = load(c + 1, (c + 1) % 2)
        scat[0].wait()
        scat[1].wait()

    return k(x, pa, pb)


def _sc_combine(o, slotA, slotB, wA, wB):
    NPAD, D = o.shape
    B = slotA.shape[0]
    TPW = B // NW
    CC = 16
    n_ch = TPW // CC
    U = 4
    mesh = plsc.VectorSubcoreMesh(core_axis_name="c", subcore_axis_name="s")

    @functools.partial(
        pl.kernel, mesh=mesh,
        out_type=jax.ShapeDtypeStruct((B, D), jnp.float32),
        scratch_types=[
            pltpu.VMEM((TPW,), jnp.int32),
            pltpu.VMEM((TPW,), jnp.int32),
            pltpu.VMEM((TPW, 16), jnp.float32),
            pltpu.VMEM((TPW, 16), jnp.float32),
            pltpu.VMEM((2, CC, D), jnp.float32),
            pltpu.VMEM((2, CC, D), jnp.float32),
            pltpu.SemaphoreType.DMA,
            pltpu.SemaphoreType.DMA,
        ],
    )
    def k(o_hbm, ia_hbm, ib_hbm, wa_hbm, wb_hbm, out_hbm,
          ia_v, ib_v, wa_v, wb_v, bufa, bufb, sema, semb):
        info = plsc.get_sparse_core_info()
        wid = lax.axis_index("s") * info.num_cores + lax.axis_index("c")
        base = wid * TPW
        pltpu.sync_copy(ia_hbm.at[pl.ds(base, TPW)], ia_v)
        pltpu.sync_copy(ib_hbm.at[pl.ds(base, TPW)], ib_v)
        pltpu.sync_copy(wa_hbm.at[pl.ds(base, TPW)], wa_v)
        pltpu.sync_copy(wb_hbm.at[pl.ds(base, TPW)], wb_v)

        def fire(c, slot):
            return (
                pltpu.async_copy(o_hbm.at[ia_v.at[pl.ds(c * CC, CC)]],
                                 bufa.at[slot], sema),
                pltpu.async_copy(o_hbm.at[ib_v.at[pl.ds(c * CC, CC)]],
                                 bufb.at[slot], semb),
            )

        cps = fire(0, 0)
        for c in range(n_ch):
            slot = c % 2
            cps[0].wait()
            cps[1].wait()
            if c + 1 < n_ch:
                cps = fire(c + 1, (c + 1) % 2)

            def add_row(i, carry2):
                wa = wa_v[c * CC + i, pl.ds(0, 16)]
                wb = wb_v[c * CC + i, pl.ds(0, 16)]

                def add_blk(j, carry3):
                    off = j * (16 * U)
                    for u in range(U):
                        sl = pl.ds(off + 16 * u, 16)
                        bufa[slot, i, sl] = (wa * bufa[slot, i, sl]
                                             + wb * bufb[slot, i, sl])
                    return carry3
                return lax.fori_loop(0, D // (16 * U), add_blk, carry2)

            lax.fori_loop(0, CC, add_row, 0)
            pltpu.sync_copy(bufa.at[slot], out_hbm.at[pl.ds(base + c * CC, CC)])

        return None

    return k(o, slotA, slotB, wA, wB)


def kernel(x, Wg, bg, W1, b1, W2, b2):
    B, D = x.shape
    E = Wg.shape[0]
    BK = 2 * B
    NT = (BK + E * (TM - 1) + TM - 1) // TM
    NPAD = NT * TM

    pa, pb, wa, wb, te, tv, xb, ll, il = _gating(x, Wg, bg, NT)

    xs = _sc_dispatch(x, pa, pb, NPAD)
    o = _gmm(te, tv, xb, xs, W1, b1, W2, b2)
    out = _sc_combine(o, pa, pb, wa, wb)

    return out, ll[0, 0], il[0, 0]
